# baseline (speedup 1.0000x reference)
import functools

import numpy as np
import jax
import jax.numpy as jnp
from jax import lax
from jax.experimental import pallas as pl
from jax.experimental.pallas import tpu as pltpu
from jax.experimental.pallas import tpu_sc as plsc


def _sinusoidal_pe(max_len, d):
    pos = np.arange(max_len, dtype=np.float32)[:, None]
    div = np.exp(np.arange(0, d, 2, dtype=np.float32) * (-np.log(10000.0) / d))
    pe = np.zeros((max_len, d), dtype=np.float32)
    pe[:, 0::2] = np.sin(pos * div)
    pe[:, 1::2] = np.cos(pos * div)
    return pe


def kernel(input, token_table):
    B, S = input.shape
    V, E = token_table.shape
    NC, NS = 2, 16
    NW = NC * NS
    BS = B * S
    n_per_w = BS // NW
    seq_per_w = n_per_w // S
    half = S // 2
    L = 16

    ids = input.astype(jnp.int32).reshape(NW, 2 * seq_per_w, half)
    pe = jnp.asarray(_sinusoidal_pe(S, E))

    mesh = plsc.VectorSubcoreMesh(core_axis_name="c", subcore_axis_name="s")

    @functools.partial(
        pl.kernel,
        out_type=jax.ShapeDtypeStruct((BS, E), jnp.float32),
        mesh=mesh,
        compiler_params=pltpu.CompilerParams(use_tc_tiling_on_sc=False),
        scratch_types=[
            pltpu.VMEM((2 * seq_per_w, half), jnp.int32),
            pltpu.VMEM((S, E), jnp.float32),
            pltpu.VMEM((S, E), jnp.float32),
            pltpu.SemaphoreType.DMA,
        ],
    )
    def run(table_hbm, ids_hbm, pe_hbm, out_hbm, idx_v, pe_v, buf_v, sem):
        wid = lax.axis_index("s") * NC + lax.axis_index("c")
        base = wid * n_per_w
        pltpu.sync_copy(ids_hbm.at[wid], idx_v)
        pltpu.sync_copy(pe_hbm, pe_v)

        def per_seq(g, carry):
            pltpu.async_copy(
                table_hbm.at[idx_v.at[2 * g]],
                buf_v.at[pl.ds(0, half)], sem).wait()
            pltpu.async_copy(
                table_hbm.at[idx_v.at[2 * g + 1]],
                buf_v.at[pl.ds(half, half)], sem).wait()

            def per_row(r, c2):
                for c in range(E // L):
                    sl = pl.ds(c * L, L)
                    buf_v[r, sl] = buf_v[r, sl] + pe_v[r, sl]
                return c2

            lax.fori_loop(0, S, per_row, 0)
            pltpu.sync_copy(buf_v, out_hbm.at[pl.ds(base + g * S, S)])
            return carry

        lax.fori_loop(0, seq_per_w, per_seq, 0)

    out = run(token_table, ids, pe)
    return out.reshape(B, S, E)


# speedup vs baseline: 1.2075x; 1.2075x over previous
import functools

import numpy as np
import jax
import jax.numpy as jnp
from jax import lax
from jax.experimental import pallas as pl
from jax.experimental.pallas import tpu as pltpu
from jax.experimental.pallas import tpu_sc as plsc


def _sinusoidal_pe(max_len, d):
    pos = np.arange(max_len, dtype=np.float32)[:, None]
    div = np.exp(np.arange(0, d, 2, dtype=np.float32) * (-np.log(10000.0) / d))
    pe = np.zeros((max_len, d), dtype=np.float32)
    pe[:, 0::2] = np.sin(pos * div)
    pe[:, 1::2] = np.cos(pos * div)
    return pe


def kernel(input, token_table):
    B, S = input.shape
    V, E = token_table.shape
    NC, NS = 2, 16
    NW = NC * NS
    BS = B * S
    n_per_w = BS // NW
    seq_per_w = n_per_w // S
    half = S // 2
    L = 16
    NBUF = 4
    LOOK = 2

    ids = input.astype(jnp.int32).reshape(NW, 2 * seq_per_w, half)
    pe = jnp.asarray(_sinusoidal_pe(S, E))

    mesh = plsc.VectorSubcoreMesh(core_axis_name="c", subcore_axis_name="s")

    @functools.partial(
        pl.kernel,
        out_type=jax.ShapeDtypeStruct((BS, E), jnp.float32),
        mesh=mesh,
        compiler_params=pltpu.CompilerParams(use_tc_tiling_on_sc=False),
        scratch_types=[
            pltpu.VMEM((2 * seq_per_w, half), jnp.int32),
            pltpu.VMEM((S, E), jnp.float32),
            pltpu.VMEM((NBUF, S, E), jnp.float32),
        ] + [pltpu.SemaphoreType.DMA] * (2 * NBUF),
    )
    def run(table_hbm, ids_hbm, pe_hbm, out_hbm, idx_v, pe_v, buf_v, *sems):
        gsems = sems[:NBUF]
        ssems = sems[NBUF:]
        wid = lax.axis_index("s") * NC + lax.axis_index("c")
        base = wid * n_per_w
        pltpu.sync_copy(ids_hbm.at[wid], idx_v)
        pltpu.sync_copy(pe_hbm, pe_v)

        def gather(g, b, h):
            return pltpu.make_async_copy(
                table_hbm.at[idx_v.at[2 * g + h]],
                buf_v.at[b].at[pl.ds(h * half, half)],
                gsems[b])

        def scatter(g, b):
            return pltpu.make_async_copy(
                buf_v.at[b],
                out_hbm.at[pl.ds(base + g * S, S)],
                ssems[b])

        for g0 in range(LOOK):
            gather(g0, g0, 0).start()
            gather(g0, g0, 1).start()

        def outer(i, carry):
            for b in range(NBUF):
                g = i * NBUF + b
                gather(g, b, 0).wait()
                gather(g, b, 1).wait()

                def add_row(r, c2):
                    for c in range(E // L):
                        sl = pl.ds(c * L, L)
                        buf_v[b, r, sl] = buf_v[b, r, sl] + pe_v[r, sl]
                    return c2

                lax.fori_loop(0, S, add_row, 0)
                scatter(g, b).start()

                gn = g + LOOK
                nb = (b + LOOK) % NBUF

                @pl.when(gn < seq_per_w)
                def _():
                    @pl.when(gn >= NBUF)
                    def _():
                        scatter(gn - NBUF, nb).wait()
                    gather(gn, nb, 0).start()
                    gather(gn, nb, 1).start()
            return carry

        lax.fori_loop(0, seq_per_w // NBUF, outer, 0)

        for g0 in range(seq_per_w - NBUF, seq_per_w):
            scatter(g0, g0 % NBUF).wait()

    out = run(token_table, ids, pe)
    return out.reshape(B, S, E)
